# baseline (speedup 1.0000x reference)
import jax
import jax.numpy as jnp
from jax.experimental import pallas as pl
from jax.experimental.pallas import tpu as pltpu


def _net_kernel(x16_ref, w1_ref, b1_ref, w2_ref, b2_ref,
                f1_ref, f1b_ref, f2_ref, f2b_ref, o_ref):
    tb = x16_ref.shape[2]

    a = jnp.dot(x16_ref[...].reshape(169 * tb, 16), w1_ref[...],
                preferred_element_type=jnp.float32)
    y1 = jnp.maximum(a + b1_ref[...], 0.0)
    y1 = y1.astype(jnp.bfloat16).reshape(13, 13, tb, 128)

    mrows = 144 * tb
    z = jnp.zeros((4 * mrows, 64), jnp.float32)
    for di in range(3):
        for dj in range(3):
            k = 3 * di + dj
            parts = []
            for ca in range(2):
                r, i0 = (ca + di) % 2, (ca + di) // 2
                for cb in range(2):
                    s, j0 = (cb + dj) % 2, (cb + dj) // 2
                    off = 32 * (2 * r + s)
                    parts.append(
                        y1[i0:i0 + 12, j0:j0 + 12, :, off:off + 32][None])
            lhs = jnp.concatenate(parts, axis=0).reshape(4 * mrows, 32)
            z = z + jnp.dot(lhs, w2_ref[k],
                            preferred_element_type=jnp.float32)
    zc = z.reshape(4, mrows, 64)
    zp = jnp.maximum(jnp.maximum(zc[0], zc[1]), jnp.maximum(zc[2], zc[3]))
    y2 = jnp.maximum(zp + b2_ref[...], 0.0)

    y2r = y2.astype(jnp.bfloat16).reshape(144, tb, 64)
    flat = jnp.concatenate([y2r[p] for p in range(144)], axis=-1)
    h = jnp.dot(flat, f1_ref[...],
                preferred_element_type=jnp.float32)
    h = jnp.maximum(h + f1b_ref[...], 0.0)

    zz = jnp.dot(h, f2_ref[...],
                 preferred_element_type=jnp.float32) + f2b_ref[...]
    m = jnp.max(zz, axis=-1, keepdims=True)
    sft = zz - m
    lse = jnp.log(jnp.sum(jnp.exp(sft), axis=-1, keepdims=True))
    o_ref[...] = (sft - lse).astype(o_ref.dtype)


def _build_w1x(conv1_w):
    zeros = jnp.zeros((32,), conv1_w.dtype)
    rows = []
    for u0 in range(2):
        for v0 in range(2):
            for r in range(2):
                for s in range(2):
                    cols = []
                    for p in range(2):
                        for q in range(2):
                            di, dj = 2 * u0 + r - p, 2 * v0 + s - q
                            if 0 <= di < 3 and 0 <= dj < 3:
                                cols.append(conv1_w[3 * di + dj, :])
                            else:
                                cols.append(zeros)
                    rows.append(jnp.concatenate(cols))
    return jnp.stack(rows)


def kernel(x_nchw, conv1_w, conv1_b, conv2_w, conv2_b,
           fc1_w, fc1_b, fc2_w, fc2_b):
    n = x_nchw.shape[0]
    x = x_nchw.reshape(n, 28, 28)
    pieces = [x[:, 2 * u0 + r: 2 * u0 + r + 26: 2,
                2 * v0 + s: 2 * v0 + s + 26: 2]
              for u0 in range(2) for v0 in range(2)
              for r in range(2) for s in range(2)]
    x16 = jnp.stack(pieces, axis=-1)
    x16 = x16.transpose(1, 2, 0, 3).astype(jnp.bfloat16)

    w1x = _build_w1x(conv1_w).astype(jnp.bfloat16)
    b1x = jnp.tile(conv1_b, (1, 4))
    w2b = conv2_w.astype(jnp.bfloat16)
    f1b16 = fc1_w.astype(jnp.bfloat16)

    tb = 16
    grid = (pl.cdiv(n, tb),)
    return pl.pallas_call(
        _net_kernel,
        out_shape=jax.ShapeDtypeStruct((n, 10), jnp.float32),
        grid_spec=pltpu.PrefetchScalarGridSpec(
            num_scalar_prefetch=0,
            grid=grid,
            in_specs=[
                pl.BlockSpec((13, 13, tb, 16), lambda i: (0, 0, i, 0)),
                pl.BlockSpec((16, 128), lambda i: (0, 0)),
                pl.BlockSpec((1, 128), lambda i: (0, 0)),
                pl.BlockSpec((9, 32, 64), lambda i: (0, 0, 0)),
                pl.BlockSpec((1, 64), lambda i: (0, 0)),
                pl.BlockSpec((9216, 128), lambda i: (0, 0)),
                pl.BlockSpec((1, 128), lambda i: (0, 0)),
                pl.BlockSpec((128, 10), lambda i: (0, 0)),
                pl.BlockSpec((1, 10), lambda i: (0, 0)),
            ],
            out_specs=pl.BlockSpec((tb, 10), lambda i: (i, 0)),
        ),
        compiler_params=pltpu.CompilerParams(
            dimension_semantics=("parallel",),
            vmem_limit_bytes=48 * 1024 * 1024),
    )(x16, w1x, b1x, w2b, conv2_b, f1b16, fc1_b, fc2_w, fc2_b)


# speedup vs baseline: 2.8997x; 2.8997x over previous
import jax
import jax.numpy as jnp
from jax.experimental import pallas as pl
from jax.experimental.pallas import tpu as pltpu


def _net_kernel(x_ref, w1_ref, b1_ref, w2_ref, b2_ref,
                f1_ref, f1b_ref, f2_ref, f2b_ref, o_ref):
    tb = x_ref.shape[2]
    x = x_ref[...]

    x16 = jnp.concatenate(
        [x[u0:u0 + 13, v0:v0 + 13, :, :]
         for u0 in range(2) for v0 in range(2)], axis=-1)
    a = jnp.dot(x16.reshape(169 * tb, 16).astype(jnp.bfloat16), w1_ref[...],
                preferred_element_type=jnp.float32)
    y1 = jnp.maximum(a + b1_ref[...], 0.0)
    y1 = y1.astype(jnp.bfloat16).reshape(13, 13, tb, 128)

    mrows = 144 * tb
    z = jnp.zeros((4 * mrows, 64), jnp.float32)
    for di in range(3):
        for dj in range(3):
            k = 3 * di + dj
            parts = []
            for ca in range(2):
                r, i0 = (ca + di) % 2, (ca + di) // 2
                for cb in range(2):
                    s, j0 = (cb + dj) % 2, (cb + dj) // 2
                    off = 32 * (2 * r + s)
                    parts.append(
                        y1[i0:i0 + 12, j0:j0 + 12, :, off:off + 32][None])
            lhs = jnp.concatenate(parts, axis=0).reshape(4 * mrows, 32)
            z = z + jnp.dot(lhs, w2_ref[k],
                            preferred_element_type=jnp.float32)
    zc = z.reshape(4, mrows, 64)
    zp = jnp.maximum(jnp.maximum(zc[0], zc[1]), jnp.maximum(zc[2], zc[3]))
    y2 = jnp.maximum(zp + b2_ref[...], 0.0)

    y2r = y2.astype(jnp.bfloat16).reshape(144, tb, 64)
    flat = jnp.concatenate([y2r[p] for p in range(144)], axis=-1)
    h = jnp.dot(flat, f1_ref[...],
                preferred_element_type=jnp.float32)
    h = jnp.maximum(h + f1b_ref[...], 0.0)

    zz = jnp.dot(h, f2_ref[...],
                 preferred_element_type=jnp.float32) + f2b_ref[...]
    m = jnp.max(zz, axis=-1, keepdims=True)
    sft = zz - m
    lse = jnp.log(jnp.sum(jnp.exp(sft), axis=-1, keepdims=True))
    o_ref[...] = (sft - lse).astype(o_ref.dtype)


def _build_w1x(conv1_w):
    zeros = jnp.zeros((32,), conv1_w.dtype)
    rows = []
    for u0 in range(2):
        for v0 in range(2):
            for r in range(2):
                for s in range(2):
                    cols = []
                    for p in range(2):
                        for q in range(2):
                            di, dj = 2 * u0 + r - p, 2 * v0 + s - q
                            if 0 <= di < 3 and 0 <= dj < 3:
                                cols.append(conv1_w[3 * di + dj, :])
                            else:
                                cols.append(zeros)
                    rows.append(jnp.concatenate(cols))
    return jnp.stack(rows)


def kernel(x_nchw, conv1_w, conv1_b, conv2_w, conv2_b,
           fc1_w, fc1_b, fc2_w, fc2_b):
    n = x_nchw.shape[0]
    x = x_nchw.reshape(n, 28, 28)
    x_par = (x.reshape(n, 14, 2, 14, 2)
             .transpose(1, 3, 0, 2, 4).reshape(14, 14, n, 4))
    w1x = _build_w1x(conv1_w).astype(jnp.bfloat16)
    b1x = jnp.tile(conv1_b, (1, 4))
    w2b = conv2_w.astype(jnp.bfloat16)
    f1b16 = fc1_w.astype(jnp.bfloat16)

    tb = 32
    grid = (pl.cdiv(n, tb),)
    return pl.pallas_call(
        _net_kernel,
        out_shape=jax.ShapeDtypeStruct((n, 10), jnp.float32),
        grid_spec=pltpu.PrefetchScalarGridSpec(
            num_scalar_prefetch=0,
            grid=grid,
            in_specs=[
                pl.BlockSpec((14, 14, tb, 4), lambda i: (0, 0, i, 0)),
                pl.BlockSpec((16, 128), lambda i: (0, 0)),
                pl.BlockSpec((1, 128), lambda i: (0, 0)),
                pl.BlockSpec((9, 32, 64), lambda i: (0, 0, 0)),
                pl.BlockSpec((1, 64), lambda i: (0, 0)),
                pl.BlockSpec((9216, 128), lambda i: (0, 0)),
                pl.BlockSpec((1, 128), lambda i: (0, 0)),
                pl.BlockSpec((128, 10), lambda i: (0, 0)),
                pl.BlockSpec((1, 10), lambda i: (0, 0)),
            ],
            out_specs=pl.BlockSpec((tb, 10), lambda i: (i, 0)),
        ),
        compiler_params=pltpu.CompilerParams(
            dimension_semantics=("parallel",),
            vmem_limit_bytes=58 * 1024 * 1024),
    )(x_par, w1x, b1x, w2b, conv2_b, f1b16, fc1_b, fc2_w, fc2_b)


# speedup vs baseline: 5.1107x; 1.7625x over previous
import jax
import jax.numpy as jnp
from jax.experimental import pallas as pl
from jax.experimental.pallas import tpu as pltpu


def _net_kernel(x_ref, w1_ref, b1_ref, w2_ref, b2_ref,
                f1_ref, f1b_ref, f2_ref, f2b_ref, o_ref):
    tb = x_ref.shape[2]
    x = x_ref[...]

    x16 = jnp.concatenate(
        [x[u0:u0 + 13, v0:v0 + 13, :, :]
         for u0 in range(2) for v0 in range(2)], axis=-1)
    a = jnp.dot(x16.reshape(169 * tb, 16).astype(jnp.bfloat16), w1_ref[...],
                preferred_element_type=jnp.float32)
    y1 = jnp.maximum(a + b1_ref[...], 0.0)
    y1 = y1.astype(jnp.bfloat16).reshape(13, 13, tb, 128)

    mrows = 144 * tb
    zs = []
    for ci in range(4):
        acc = None
        for (i0, j0, l0, kk, ro) in _SEGS[ci]:
            d = jnp.dot(
                y1[i0:i0 + 12, j0:j0 + 12, :, l0:l0 + kk].reshape(mrows, kk),
                w2_ref[ci, ro:ro + kk, :],
                preferred_element_type=jnp.float32)
            acc = d if acc is None else acc + d
        zs.append(acc)
    zp = jnp.maximum(jnp.maximum(zs[0], zs[1]), jnp.maximum(zs[2], zs[3]))
    y2 = jnp.maximum(zp + b2_ref[...], 0.0)

    y2r = y2.astype(jnp.bfloat16).reshape(144, tb, 64)
    flat = jnp.concatenate([y2r[p] for p in range(144)], axis=-1)
    h = jnp.dot(flat, f1_ref[...],
                preferred_element_type=jnp.float32)
    h = jnp.maximum(h + f1b_ref[...], 0.0)

    zz = jnp.dot(h, f2_ref[...],
                 preferred_element_type=jnp.float32) + f2b_ref[...]
    m = jnp.max(zz, axis=-1, keepdims=True)
    sft = zz - m
    lse = jnp.log(jnp.sum(jnp.exp(sft), axis=-1, keepdims=True))
    o_ref[...] = (sft - lse).astype(o_ref.dtype)


def _corner_segments():
    plans = []
    for a in range(2):
        for b in range(2):
            segs, rowoff = [], 0
            for i0 in range(2):
                for j0 in range(2):
                    rs = [r for r in range(2) if 0 <= 2 * i0 + r - a <= 2]
                    ss = [s for s in range(2) if 0 <= 2 * j0 + s - b <= 2]
                    blocks = sorted(2 * r + s for r in rs for s in ss)
                    runs, run = [], [blocks[0]]
                    for t in blocks[1:]:
                        if t == run[-1] + 1:
                            run.append(t)
                        else:
                            runs.append(run)
                            run = [t]
                    runs.append(run)
                    for run in runs:
                        kk = 32 * len(run)
                        segs.append((i0, j0, 32 * run[0], kk, rowoff, run))
                        rowoff += kk
            plans.append(segs)
    return plans


_PLANS = _corner_segments()
_SEGS = [[(i0, j0, l0, kk, ro) for (i0, j0, l0, kk, ro, _) in p]
         for p in _PLANS]


def _build_w2g(conv2_w):
    corners = []
    for ci, (a, b) in enumerate(((0, 0), (0, 1), (1, 0), (1, 1))):
        rows = []
        for (i0, j0, l0, kk, ro, run) in _PLANS[ci]:
            for t in run:
                r, s = t // 2, t % 2
                di, dj = 2 * i0 + r - a, 2 * j0 + s - b
                rows.append(conv2_w[3 * di + dj])
        corners.append(jnp.concatenate(rows, axis=0))
    return jnp.stack(corners)


def _build_w1x(conv1_w):
    zeros = jnp.zeros((32,), conv1_w.dtype)
    rows = []
    for u0 in range(2):
        for v0 in range(2):
            for r in range(2):
                for s in range(2):
                    cols = []
                    for p in range(2):
                        for q in range(2):
                            di, dj = 2 * u0 + r - p, 2 * v0 + s - q
                            if 0 <= di < 3 and 0 <= dj < 3:
                                cols.append(conv1_w[3 * di + dj, :])
                            else:
                                cols.append(zeros)
                    rows.append(jnp.concatenate(cols))
    return jnp.stack(rows)


def kernel(x_nchw, conv1_w, conv1_b, conv2_w, conv2_b,
           fc1_w, fc1_b, fc2_w, fc2_b):
    n = x_nchw.shape[0]
    x = x_nchw.reshape(n, 28, 28)
    x_par = (x.reshape(n, 14, 2, 14, 2)
             .transpose(1, 3, 0, 2, 4).reshape(14, 14, n, 4))
    w1x = _build_w1x(conv1_w).astype(jnp.bfloat16)
    b1x = jnp.tile(conv1_b, (1, 4))
    w2b = _build_w2g(conv2_w).astype(jnp.bfloat16)
    f1b16 = fc1_w.astype(jnp.bfloat16)

    tb = 32
    grid = (pl.cdiv(n, tb),)
    return pl.pallas_call(
        _net_kernel,
        out_shape=jax.ShapeDtypeStruct((n, 10), jnp.float32),
        grid_spec=pltpu.PrefetchScalarGridSpec(
            num_scalar_prefetch=0,
            grid=grid,
            in_specs=[
                pl.BlockSpec((14, 14, tb, 4), lambda i: (0, 0, i, 0)),
                pl.BlockSpec((16, 128), lambda i: (0, 0)),
                pl.BlockSpec((1, 128), lambda i: (0, 0)),
                pl.BlockSpec((4, 288, 64), lambda i: (0, 0, 0)),
                pl.BlockSpec((1, 64), lambda i: (0, 0)),
                pl.BlockSpec((9216, 128), lambda i: (0, 0)),
                pl.BlockSpec((1, 128), lambda i: (0, 0)),
                pl.BlockSpec((128, 10), lambda i: (0, 0)),
                pl.BlockSpec((1, 10), lambda i: (0, 0)),
            ],
            out_specs=pl.BlockSpec((tb, 10), lambda i: (i, 0)),
        ),
        compiler_params=pltpu.CompilerParams(
            dimension_semantics=("parallel",),
            vmem_limit_bytes=58 * 1024 * 1024),
    )(x_par, w1x, b1x, w2b, conv2_b, f1b16, fc1_b, fc2_w, fc2_b)


# speedup vs baseline: 5.2599x; 1.0292x over previous
import jax
import jax.numpy as jnp
from jax.experimental import pallas as pl
from jax.experimental.pallas import tpu as pltpu


def _net_kernel(x_ref, w1_ref, b1_ref, w2_ref, b2_ref,
                f1_ref, f1b_ref, f2_ref, f2b_ref, o_ref):
    tb = x_ref.shape[2]
    x = x_ref[...]

    x16 = jnp.concatenate(
        [x[u0:u0 + 13, v0:v0 + 13, :, :]
         for u0 in range(2) for v0 in range(2)], axis=-1)
    a = jnp.dot(x16.reshape(169 * tb, 16).astype(jnp.bfloat16), w1_ref[...],
                preferred_element_type=jnp.float32)
    y1 = jnp.maximum(a.astype(jnp.bfloat16) + b1_ref[...], jnp.bfloat16(0))
    y1 = y1.reshape(13, 13, tb, 128)

    mrows = 144 * tb
    zs = []
    for ci in range(4):
        acc = None
        for (i0, j0, l0, kk, ro) in _SEGS[ci]:
            d = jnp.dot(
                y1[i0:i0 + 12, j0:j0 + 12, :, l0:l0 + kk].reshape(mrows, kk),
                w2_ref[ci, ro:ro + kk, :],
                preferred_element_type=jnp.float32)
            acc = d if acc is None else acc + d
        zs.append(acc)
    zp = jnp.maximum(jnp.maximum(zs[0], zs[1]), jnp.maximum(zs[2], zs[3]))
    y2 = jnp.maximum(zp + b2_ref[...], 0.0)

    y2r = y2.astype(jnp.bfloat16).reshape(144, tb, 64)
    flat = jnp.concatenate([y2r[p] for p in range(144)], axis=-1)
    h = jnp.dot(flat, f1_ref[...],
                preferred_element_type=jnp.float32)
    h = jnp.maximum(h + f1b_ref[...], 0.0)

    zz = jnp.dot(h, f2_ref[...],
                 preferred_element_type=jnp.float32) + f2b_ref[...]
    m = jnp.max(zz, axis=-1, keepdims=True)
    sft = zz - m
    lse = jnp.log(jnp.sum(jnp.exp(sft), axis=-1, keepdims=True))
    o_ref[...] = (sft - lse).astype(o_ref.dtype)


def _corner_segments():
    plans = []
    for a in range(2):
        for b in range(2):
            segs, rowoff = [], 0
            for i0 in range(2):
                for j0 in range(2):
                    rs = [r for r in range(2) if 0 <= 2 * i0 + r - a <= 2]
                    ss = [s for s in range(2) if 0 <= 2 * j0 + s - b <= 2]
                    blocks = sorted(2 * r + s for r in rs for s in ss)
                    runs, run = [], [blocks[0]]
                    for t in blocks[1:]:
                        if t == run[-1] + 1:
                            run.append(t)
                        else:
                            runs.append(run)
                            run = [t]
                    runs.append(run)
                    for run in runs:
                        kk = 32 * len(run)
                        segs.append((i0, j0, 32 * run[0], kk, rowoff, run))
                        rowoff += kk
            plans.append(segs)
    return plans


_PLANS = _corner_segments()
_SEGS = [[(i0, j0, l0, kk, ro) for (i0, j0, l0, kk, ro, _) in p]
         for p in _PLANS]


def _build_w2g(conv2_w):
    corners = []
    for ci, (a, b) in enumerate(((0, 0), (0, 1), (1, 0), (1, 1))):
        rows = []
        for (i0, j0, l0, kk, ro, run) in _PLANS[ci]:
            for t in run:
                r, s = t // 2, t % 2
                di, dj = 2 * i0 + r - a, 2 * j0 + s - b
                rows.append(conv2_w[3 * di + dj])
        corners.append(jnp.concatenate(rows, axis=0))
    return jnp.stack(corners)


def _build_w1x(conv1_w):
    zeros = jnp.zeros((32,), conv1_w.dtype)
    rows = []
    for u0 in range(2):
        for v0 in range(2):
            for r in range(2):
                for s in range(2):
                    cols = []
                    for p in range(2):
                        for q in range(2):
                            di, dj = 2 * u0 + r - p, 2 * v0 + s - q
                            if 0 <= di < 3 and 0 <= dj < 3:
                                cols.append(conv1_w[3 * di + dj, :])
                            else:
                                cols.append(zeros)
                    rows.append(jnp.concatenate(cols))
    return jnp.stack(rows)


def kernel(x_nchw, conv1_w, conv1_b, conv2_w, conv2_b,
           fc1_w, fc1_b, fc2_w, fc2_b):
    n = x_nchw.shape[0]
    x = x_nchw.reshape(n, 28, 28)
    x_par = (x.reshape(n, 14, 2, 14, 2)
             .transpose(1, 3, 0, 2, 4).reshape(14, 14, n, 4))
    w1x = _build_w1x(conv1_w).astype(jnp.bfloat16)
    b1x = jnp.tile(conv1_b, (1, 4)).astype(jnp.bfloat16)
    w2b = _build_w2g(conv2_w).astype(jnp.bfloat16)
    f1b16 = fc1_w.astype(jnp.bfloat16)

    tb = 64
    grid = (pl.cdiv(n, tb),)
    return pl.pallas_call(
        _net_kernel,
        out_shape=jax.ShapeDtypeStruct((n, 10), jnp.float32),
        grid_spec=pltpu.PrefetchScalarGridSpec(
            num_scalar_prefetch=0,
            grid=grid,
            in_specs=[
                pl.BlockSpec((14, 14, tb, 4), lambda i: (0, 0, i, 0)),
                pl.BlockSpec((16, 128), lambda i: (0, 0)),
                pl.BlockSpec((1, 128), lambda i: (0, 0)),
                pl.BlockSpec((4, 288, 64), lambda i: (0, 0, 0)),
                pl.BlockSpec((1, 64), lambda i: (0, 0)),
                pl.BlockSpec((9216, 128), lambda i: (0, 0)),
                pl.BlockSpec((1, 128), lambda i: (0, 0)),
                pl.BlockSpec((128, 10), lambda i: (0, 0)),
                pl.BlockSpec((1, 10), lambda i: (0, 0)),
            ],
            out_specs=pl.BlockSpec((tb, 10), lambda i: (i, 0)),
        ),
        compiler_params=pltpu.CompilerParams(
            dimension_semantics=("parallel",),
            vmem_limit_bytes=58 * 1024 * 1024),
    )(x_par, w1x, b1x, w2b, conv2_b, f1b16, fc1_b, fc2_w, fc2_b)
